# Initial kernel scaffold; baseline (speedup 1.0000x reference)
import jax
import jax.numpy as jnp
from jax.experimental import pallas as pl


def kernel(input_ids, input_mask, word_table, pos_table, type_table, gamma, beta):
    raise NotImplementedError("write your pallas kernel here")


# speedup vs baseline: 1.3046x; 1.3046x over previous
import functools

import jax
import jax.numpy as jnp
from jax import lax
from jax.experimental import pallas as pl
from jax.experimental.pallas import tpu as pltpu
from jax.experimental.pallas import tpu_sc as plsc

HIDDEN = 64
TYPE_VOCAB = 5
B = 4096
L = 50
EPS = 1e-12

NC = 2
NS = 16
LANES = 16
NW = NC * NS
N = B * L
ROWS_PER_W = N // NW
C = 640
NCHUNK = ROWS_PER_W // C
GCHUNK = 128
NG = C // GCHUNK
GROUPS = C // LANES
POS_STAGE = 56


def _rsqrt(v):
    i = plsc.bitcast(v, jnp.int32)
    i = jnp.int32(0x5F3759DF) - (i >> 1)
    y = plsc.bitcast(i, jnp.float32)
    for _ in range(3):
        y = y * (1.5 - 0.5 * v * y * y)
    return y


def _body(ids_hbm, msk_hbm, word_hbm, pos_hbm, typ_hbm, gspl_hbm, bspl_hbm,
          out_hbm, idx_v, msk_v, x_v, pt_v, pos_v, typ_v, gspl_v, bspl_v, sem):
    wid = lax.axis_index("s") * NC + lax.axis_index("c")
    base0 = wid * ROWS_PER_W

    pltpu.sync_copy(pos_hbm.at[pl.ds(0, POS_STAGE)], pos_v)
    pltpu.sync_copy(typ_hbm, typ_v)
    pltpu.sync_copy(gspl_hbm, gspl_v)
    pltpu.sync_copy(bspl_hbm, bspl_v)

    tvals = [typ_v[t, pl.ds(k * LANES, LANES)]
             for t in range(TYPE_VOCAB) for k in range(HIDDEN // LANES)]

    def _pt_body(l, carry):
        for k in range(HIDDEN // LANES):
            pv = pos_v[l, pl.ds(k * LANES, LANES)]
            for t in range(TYPE_VOCAB):
                pt_v[l * TYPE_VOCAB + t, pl.ds(k * LANES, LANES)] = (
                    pv + tvals[t * (HIDDEN // LANES) + k])
        return carry

    lax.fori_loop(0, L, _pt_body, 0)

    iota = lax.iota(jnp.int32, LANES)

    def _chunk(c, carry):
        base = base0 + c * C
        pltpu.sync_copy(ids_hbm.at[pl.ds(base, C)], idx_v)
        pltpu.sync_copy(msk_hbm.at[pl.ds(base, C)], msk_v)
        pltpu.async_copy(word_hbm.at[idx_v], x_v, sem).wait()

        def _group(g, inner):
            i0 = g * LANES
            rvec = i0 + iota
            gvec = base + rvec
            lvec = gvec % L
            tvec = msk_v[pl.ds(i0, LANES)]
            cvec = lvec * TYPE_VOCAB + tvec
            s = jnp.zeros((LANES,), jnp.float32)
            q = jnp.zeros((LANES,), jnp.float32)
            for h in range(HIDDEN):
                hvec = jnp.full((LANES,), h, jnp.int32)
                xh = plsc.load_gather(x_v, [rvec, hvec])
                ph = plsc.load_gather(pt_v, [cvec, hvec])
                v = xh + ph
                plsc.store_scatter(x_v, [rvec, hvec], v)
                s = s + v
                q = q + v * v
            mean = s * (1.0 / HIDDEN)
            var = q * (1.0 / HIDDEN) - mean * mean
            r = _rsqrt(var + EPS)
            for h in range(HIDDEN):
                hvec = jnp.full((LANES,), h, jnp.int32)
                v = plsc.load_gather(x_v, [rvec, hvec])
                y = (v - mean) * r * gspl_v[h] + bspl_v[h]
                plsc.store_scatter(x_v, [rvec, hvec], y)
            return inner

        lax.fori_loop(0, GROUPS, _group, 0)
        pltpu.sync_copy(x_v, out_hbm.at[pl.ds(base, C)])
        return carry

    lax.fori_loop(0, NCHUNK, _chunk, 0)


def kernel(input_ids, input_mask, word_table, pos_table, type_table, gamma, beta):
    ids = input_ids.reshape(N)
    msk = input_mask.reshape(N)
    gspl = jnp.broadcast_to(gamma[:, None], (HIDDEN, LANES))
    bspl = jnp.broadcast_to(beta[:, None], (HIDDEN, LANES))

    mesh = plsc.VectorSubcoreMesh(
        core_axis_name="c", subcore_axis_name="s",
        num_cores=NC, num_subcores=NS)
    f = pl.kernel(
        _body,
        out_type=jax.ShapeDtypeStruct((N, HIDDEN), jnp.float32),
        mesh=mesh,
        compiler_params=pltpu.CompilerParams(
            needs_layout_passes=False, use_tc_tiling_on_sc=False),
        scratch_types=[
            pltpu.VMEM((C,), jnp.int32),
            pltpu.VMEM((C,), jnp.int32),
            pltpu.VMEM((C, HIDDEN), jnp.float32),
            pltpu.VMEM((L * TYPE_VOCAB, HIDDEN), jnp.float32),
            pltpu.VMEM((POS_STAGE, HIDDEN), jnp.float32),
            pltpu.VMEM((TYPE_VOCAB, HIDDEN), jnp.float32),
            pltpu.VMEM((HIDDEN, LANES), jnp.float32),
            pltpu.VMEM((HIDDEN, LANES), jnp.float32),
            pltpu.SemaphoreType.DMA,
        ],
    )
    out = f(ids, msk, word_table, pos_table, type_table, gspl, bspl)
    return out.reshape(B, L, HIDDEN)


# speedup vs baseline: 2.5752x; 1.9738x over previous
import jax
import jax.numpy as jnp
from jax import lax
from jax.experimental import pallas as pl
from jax.experimental.pallas import tpu as pltpu
from jax.experimental.pallas import tpu_sc as plsc

HIDDEN = 64
KV = HIDDEN // 16
TYPE_VOCAB = 5
B = 4096
L = 50
EPS = 1e-12

NC = 2
NS = 16
LANES = 16
NW = NC * NS
N = B * L
ROWS_PER_W = N // NW
C = 640
NCHUNK = ROWS_PER_W // C
GROUP = 16
POS_STAGE = 56


def _take(vec, idx):
    return lax.gather(
        vec, idx[:, None],
        dimension_numbers=lax.GatherDimensionNumbers(
            offset_dims=(), collapsed_slice_dims=(0,), start_index_map=(0,)),
        slice_sizes=(1,),
        mode=lax.GatherScatterMode.PROMISE_IN_BOUNDS)


def _rsqrt(v):
    i = plsc.bitcast(v, jnp.int32)
    i = jnp.int32(0x5F3759DF) - (i >> 1)
    y = plsc.bitcast(i, jnp.float32)
    for _ in range(3):
        y = y * (1.5 - 0.5 * v * y * y)
    return y


def _body(ids_hbm, msk_hbm, word_hbm, pos_hbm, typ_hbm, gam_hbm, bet_hbm,
          out_hbm, idx_v, msk_v, x_v, y_v, pos_v, typ_v, gam_v, bet_v, sem):
    wid = lax.axis_index("s") * NC + lax.axis_index("c")
    base0 = wid * ROWS_PER_W

    pltpu.sync_copy(pos_hbm.at[pl.ds(0, POS_STAGE)], pos_v)
    pltpu.sync_copy(typ_hbm, typ_v)
    pltpu.sync_copy(gam_hbm, gam_v)
    pltpu.sync_copy(bet_hbm, bet_v)

    tv = [[typ_v[t, pl.ds(k * LANES, LANES)] for k in range(KV)]
          for t in range(TYPE_VOCAB)]
    gv = [gam_v[pl.ds(k * LANES, LANES)] for k in range(KV)]
    bv = [bet_v[pl.ds(k * LANES, LANES)] for k in range(KV)]
    idx15 = jnp.full((LANES,), LANES - 1, jnp.int32)

    def _chunk(c, carry):
        base = base0 + c * C
        pltpu.sync_copy(ids_hbm.at[pl.ds(base, C)], idx_v)
        pltpu.sync_copy(msk_hbm.at[pl.ds(base, C)], msk_v)
        pltpu.async_copy(word_hbm.at[idx_v], x_v, sem).wait()

        @plsc.parallel_loop(0, C, GROUP)
        def _group(i):
            tvec = msk_v[pl.ds(i, GROUP)]
            g0 = base + i
            for j in range(GROUP):
                row = i + j
                lpos = lax.rem(g0 + j, L)
                xs = [x_v[row, pl.ds(k * LANES, LANES)] for k in range(KV)]
                ps = [pos_v[lpos, pl.ds(k * LANES, LANES)] for k in range(KV)]
                tsp = _take(tvec, jnp.full((LANES,), j, jnp.int32))
                m = [tsp == t for t in range(TYPE_VOCAB - 1)]
                vs = []
                for k in range(KV):
                    tk = tv[TYPE_VOCAB - 1][k]
                    for t in range(TYPE_VOCAB - 2, -1, -1):
                        tk = jnp.where(m[t], tv[t][k], tk)
                    vs.append(xs[k] + ps[k] + tk)
                s = (vs[0] + vs[1]) + (vs[2] + vs[3])
                q = (vs[0] * vs[0] + vs[1] * vs[1]) + (
                    vs[2] * vs[2] + vs[3] * vs[3])
                tot = _take(plsc.cumsum(s), idx15)
                totq = _take(plsc.cumsum(q), idx15)
                mean = tot * (1.0 / HIDDEN)
                var = totq * (1.0 / HIDDEN) - mean * mean
                r = _rsqrt(var + EPS)
                for k in range(KV):
                    y_v[row, pl.ds(k * LANES, LANES)] = (
                        (vs[k] - mean) * r * gv[k] + bv[k])

        pltpu.sync_copy(y_v, out_hbm.at[pl.ds(base, C)])
        return carry

    lax.fori_loop(0, NCHUNK, _chunk, 0)


def kernel(input_ids, input_mask, word_table, pos_table, type_table, gamma, beta):
    ids = input_ids.reshape(N)
    msk = input_mask.reshape(N)

    mesh = plsc.VectorSubcoreMesh(
        core_axis_name="c", subcore_axis_name="s",
        num_cores=NC, num_subcores=NS)
    f = pl.kernel(
        _body,
        out_type=jax.ShapeDtypeStruct((N, HIDDEN), jnp.float32),
        mesh=mesh,
        compiler_params=pltpu.CompilerParams(
            needs_layout_passes=False, use_tc_tiling_on_sc=False),
        scratch_types=[
            pltpu.VMEM((C,), jnp.int32),
            pltpu.VMEM((C,), jnp.int32),
            pltpu.VMEM((C, HIDDEN), jnp.float32),
            pltpu.VMEM((C, HIDDEN), jnp.float32),
            pltpu.VMEM((POS_STAGE, HIDDEN), jnp.float32),
            pltpu.VMEM((TYPE_VOCAB, HIDDEN), jnp.float32),
            pltpu.VMEM((HIDDEN,), jnp.float32),
            pltpu.VMEM((HIDDEN,), jnp.float32),
            pltpu.SemaphoreType.DMA,
        ],
    )
    out = f(ids, msk, word_table, pos_table, type_table, gamma, beta)
    return out.reshape(B, L, HIDDEN)
